# baseline (speedup 1.0000x reference)
import functools

import jax
import jax.numpy as jnp
from jax import lax
from jax.experimental import pallas as pl
from jax.experimental.pallas import tpu as pltpu
from jax.experimental.pallas import tpu_sc as plsc

N = 10000
E = 320000
C = 128
NG = 64

NC = 2
NS = 16
NW = NC * NS
K = 128
NCP = 80
EPW = NCP * K
REW = E // NW
WPAD = EPW - REW
NTRASH = 16
HALF = NCP // 2
RPT = 624
TAIL = N - NS * RPT

def _segment_sum_body(x_hbm, srcs_hbm, dsts_hbm, zeros_hbm, out_hbm,
                      sidx2, didx0, didx1, rows0, rows1, acc,
                      gsem0, gsem1, dsem0, dsem1):
    cid = lax.axis_index("c")
    sid = lax.axis_index("s")
    wid = sid * NC + cid

    pltpu.sync_copy(srcs_hbm.at[wid], sidx2)

    pltpu.sync_copy(zeros_hbm.at[pl.ds(sid * RPT, RPT)],
                    acc.at[pl.ds(sid * RPT, RPT)])

    @pl.when(sid == 0)
    def _():
        pltpu.sync_copy(zeros_hbm.at[pl.ds(NS * RPT, TAIL + NTRASH)],
                        acc.at[pl.ds(NS * RPT, TAIL + NTRASH)])

    plsc.subcore_barrier()

    def gather(j, buf, sem):
        pltpu.async_copy(x_hbm.at[sidx2.at[j]], buf, sem)

    def gwait(buf, sem):
        pltpu.make_async_copy(x_hbm.at[pl.ds(0, K)], buf, sem).wait()

    def dload(j, dbuf, sem):
        pltpu.async_copy(dsts_hbm.at[wid * NCP + j], dbuf, sem)

    def dwait(dbuf, sem):
        pltpu.make_async_copy(dsts_hbm.at[0], dbuf, sem).wait()

    def scatter_add(buf, dbuf):
        pltpu.sync_copy(buf, acc.at[dbuf.at[0]], add=True)

    dload(0, didx0, dsem0)
    gather(0, rows0, gsem0)

    def outer(i, carry):
        b = 2 * i + 1
        dload(b, didx1, dsem1)
        gather(b, rows1, gsem1)
        gwait(rows0, gsem0)
        dwait(didx0, dsem0)
        scatter_add(rows0, didx0)

        @pl.when(i + 1 < HALF)
        def _():
            dload(2 * i + 2, didx0, dsem0)
            gather(2 * i + 2, rows0, gsem0)

        gwait(rows1, gsem1)
        dwait(didx1, dsem1)
        scatter_add(rows1, didx1)
        return carry

    lax.fori_loop(0, HALF, outer, 0)
    plsc.subcore_barrier()

    pltpu.sync_copy(acc.at[pl.ds(sid * RPT, RPT)],
                    out_hbm.at[cid].at[pl.ds(sid * RPT, RPT)])

    @pl.when(sid == 0)
    def _():
        pltpu.sync_copy(acc.at[pl.ds(NS * RPT, TAIL)],
                        out_hbm.at[cid].at[pl.ds(NS * RPT, TAIL)])


@functools.cache
def _segment_sum_sc():
    mesh = plsc.VectorSubcoreMesh(core_axis_name="c", subcore_axis_name="s",
                                  num_cores=NC, num_subcores=NS)
    return pl.kernel(
        _segment_sum_body,
        out_type=jax.ShapeDtypeStruct((NC, N, C), jnp.float32),
        mesh=mesh,
        scratch_types=[
            pltpu.VMEM((NCP, K), jnp.int32),
            pltpu.VMEM((1, K), jnp.int32),
            pltpu.VMEM((1, K), jnp.int32),
            pltpu.VMEM((K, C), jnp.float32),
            pltpu.VMEM((K, C), jnp.float32),
            pltpu.VMEM_SHARED((N + NTRASH, C), jnp.float32),
            pltpu.SemaphoreType.DMA,
            pltpu.SemaphoreType.DMA,
            pltpu.SemaphoreType.DMA,
            pltpu.SemaphoreType.DMA,
        ],
    )


def _mlp_body(x_ref, parts_ref, wat_ref, ba_ref, g_ref, be_ref, wbt_ref,
              bb_ref, out_ref, *, relu_out):
    h = x_ref[...] + parts_ref[0] + parts_ref[1]
    h1 = jnp.dot(h, wat_ref[...], preferred_element_type=jnp.float32)
    h1 = h1 + ba_ref[...]
    m = jnp.mean(h1, axis=0, keepdims=True)
    d = h1 - m
    v = jnp.mean(d * d, axis=0, keepdims=True)
    hb = g_ref[...] * d * lax.rsqrt(v + 1e-5) + be_ref[...]
    hb = jnp.maximum(hb, 0.0)
    h2 = jnp.dot(hb, wbt_ref[...], preferred_element_type=jnp.float32)
    h2 = h2 + bb_ref[...]
    if relu_out:
        h2 = jnp.maximum(h2, 0.0)
    out_ref[...] = h2


def _mlp_tc(x, parts, Wa, ba, g, be, Wb, bb, relu_out):
    return pl.pallas_call(
        functools.partial(_mlp_body, relu_out=relu_out),
        out_shape=jax.ShapeDtypeStruct((N, C), jnp.float32),
    )(x, parts, Wa.T, ba.reshape(1, C), g.reshape(1, C), be.reshape(1, C),
      Wb.T, bb.reshape(1, C))


def _pool_body(x1_ref, x2_ref, x3_ref, batch_ref, w1_ref, w2_ref, w3_ref,
               blin_ref, out_ref):
    gids = lax.broadcasted_iota(jnp.int32, (N, NG), 1)
    onehot = jnp.where(gids == batch_ref[...], 1.0, 0.0).astype(jnp.float32)
    counts = jnp.sum(onehot, axis=0, keepdims=True)
    inv = 1.0 / jnp.maximum(counts, 1.0)

    dn = (((0,), (0,)), ((), ()))

    def seg(x_ref, w_ref):
        sums = lax.dot_general(onehot, x_ref[...], dn,
                               preferred_element_type=jnp.float32)
        pooled = sums * inv.reshape(NG, 1)
        return jnp.dot(pooled, w_ref[...], preferred_element_type=jnp.float32)

    out = seg(x1_ref, w1_ref) + seg(x2_ref, w2_ref) + seg(x3_ref, w3_ref)
    out_ref[...] = out + blin_ref[...]


def _pool_tc(x1, x2, x3, batch, Wlin, blin):
    w1 = Wlin[:, :C].T
    w2 = Wlin[:, C:2 * C].T
    w3 = Wlin[:, 2 * C:].T
    return pl.pallas_call(
        _pool_body,
        out_shape=jax.ShapeDtypeStruct((NG, C), jnp.float32),
    )(x1, x2, x3, batch.reshape(N, 1), w1, w2, w3, blin.reshape(1, C))


def kernel(x, edge_index, batch, W1a, b1a, g1, be1, W1b, b1b, W2a, b2a, g2,
           be2, W2b, b2b, W3a, b3a, g3, be3, W3b, b3b, Wlin, blin):
    src = jnp.pad(edge_index[0].reshape(NW, REW),
                  ((0, 0), (0, WPAD))).reshape(NW, NCP, K)
    trash = (N + jnp.arange(NW, dtype=jnp.int32) // NC)[:, None]
    dst = jnp.concatenate(
        [edge_index[1].reshape(NW, REW),
         jnp.broadcast_to(trash, (NW, WPAD))],
        axis=1).reshape(NW * NCP, 1, K)
    zeros = jnp.zeros((N + NTRASH, C), dtype=jnp.float32)

    seg = _segment_sum_sc()
    p1 = seg(x, src, dst, zeros)
    x1 = _mlp_tc(x, p1, W1a, b1a, g1, be1, W1b, b1b, relu_out=True)
    p2 = seg(x1, src, dst, zeros)
    x2 = _mlp_tc(x1, p2, W2a, b2a, g2, be2, W2b, b2b, relu_out=True)
    p3 = seg(x2, src, dst, zeros)
    x3 = _mlp_tc(x2, p3, W3a, b3a, g3, be3, W3b, b3b, relu_out=False)

    return _pool_tc(x1, x2, x3, batch, Wlin, blin)


# speedup vs baseline: 2.4454x; 2.4454x over previous
import functools

import jax
import jax.numpy as jnp
from jax import lax
from jax.experimental import pallas as pl
from jax.experimental.pallas import tpu as pltpu
from jax.experimental.pallas import tpu_sc as plsc

N = 10000
E = 320000
C = 128
NG = 64

NC = 2
NS = 16
NW = NC * NS
K = 80
EPW = E // NW
NCHUNK = EPW // K
NPAIR = (NCHUNK - 1) // 2
RPT = 624
TAIL = N - NS * RPT

def _segment_sum_body(x_hbm, src_hbm, dst_hbm, zeros_hbm, out_hbm,
                      sidx0, sidx1, didx0, didx1, rows0, rows1, acc,
                      isem0, isem1, gsem0, gsem1):
    cid = lax.axis_index("c")
    sid = lax.axis_index("s")
    wid = sid * NC + cid

    pltpu.sync_copy(zeros_hbm.at[pl.ds(sid * RPT, RPT)],
                    acc.at[pl.ds(sid * RPT, RPT)])

    @pl.when(sid == 0)
    def _():
        pltpu.sync_copy(zeros_hbm.at[pl.ds(NS * RPT, TAIL)],
                        acc.at[pl.ds(NS * RPT, TAIL)])

    plsc.subcore_barrier()

    base = wid * EPW

    def iload(j, sbuf, dbuf, sem):
        pltpu.async_copy(src_hbm.at[pl.ds(base + j * K, K)], sbuf, sem)
        pltpu.async_copy(dst_hbm.at[pl.ds(base + j * K, K)], dbuf, sem)

    def iwait(sbuf, dbuf, sem):
        pltpu.make_async_copy(src_hbm.at[pl.ds(0, K)], sbuf, sem).wait()
        pltpu.make_async_copy(dst_hbm.at[pl.ds(0, K)], dbuf, sem).wait()

    def gather(sbuf, buf, sem):
        pltpu.async_copy(x_hbm.at[sbuf], buf, sem)

    def gwait(buf, sem):
        pltpu.make_async_copy(x_hbm.at[pl.ds(0, K)], buf, sem).wait()

    def scatter_add(buf, dbuf):
        pltpu.sync_copy(buf, acc.at[dbuf], add=True)

    iload(0, sidx0, didx0, isem0)
    iwait(sidx0, didx0, isem0)
    gather(sidx0, rows0, gsem0)

    def outer(i, carry):
        b = 2 * i + 1
        iload(b, sidx1, didx1, isem1)
        iwait(sidx1, didx1, isem1)
        gather(sidx1, rows1, gsem1)
        gwait(rows0, gsem0)
        scatter_add(rows0, didx0)
        iload(b + 1, sidx0, didx0, isem0)
        iwait(sidx0, didx0, isem0)
        gather(sidx0, rows0, gsem0)
        gwait(rows1, gsem1)
        scatter_add(rows1, didx1)
        return carry

    lax.fori_loop(0, NPAIR, outer, 0)
    gwait(rows0, gsem0)
    scatter_add(rows0, didx0)
    plsc.subcore_barrier()

    pltpu.sync_copy(acc.at[pl.ds(sid * RPT, RPT)],
                    out_hbm.at[cid].at[pl.ds(sid * RPT, RPT)])

    @pl.when(sid == 0)
    def _():
        pltpu.sync_copy(acc.at[pl.ds(NS * RPT, TAIL)],
                        out_hbm.at[cid].at[pl.ds(NS * RPT, TAIL)])


@functools.cache
def _segment_sum_sc():
    mesh = plsc.VectorSubcoreMesh(core_axis_name="c", subcore_axis_name="s",
                                  num_cores=NC, num_subcores=NS)
    return pl.kernel(
        _segment_sum_body,
        out_type=jax.ShapeDtypeStruct((NC, N, C), jnp.float32),
        mesh=mesh,
        scratch_types=[
            pltpu.VMEM((K,), jnp.int32),
            pltpu.VMEM((K,), jnp.int32),
            pltpu.VMEM((K,), jnp.int32),
            pltpu.VMEM((K,), jnp.int32),
            pltpu.VMEM((K, C), jnp.float32),
            pltpu.VMEM((K, C), jnp.float32),
            pltpu.VMEM_SHARED((N, C), jnp.float32),
            pltpu.SemaphoreType.DMA,
            pltpu.SemaphoreType.DMA,
            pltpu.SemaphoreType.DMA,
            pltpu.SemaphoreType.DMA,
        ],
    )


def _mlp_body(x_ref, parts_ref, wat_ref, ba_ref, g_ref, be_ref, wbt_ref,
              bb_ref, out_ref, *, relu_out):
    h = x_ref[...] + parts_ref[0] + parts_ref[1]
    h1 = jnp.dot(h, wat_ref[...], preferred_element_type=jnp.float32)
    h1 = h1 + ba_ref[...]
    m = jnp.mean(h1, axis=0, keepdims=True)
    d = h1 - m
    v = jnp.mean(d * d, axis=0, keepdims=True)
    hb = g_ref[...] * d * lax.rsqrt(v + 1e-5) + be_ref[...]
    hb = jnp.maximum(hb, 0.0)
    h2 = jnp.dot(hb, wbt_ref[...], preferred_element_type=jnp.float32)
    h2 = h2 + bb_ref[...]
    if relu_out:
        h2 = jnp.maximum(h2, 0.0)
    out_ref[...] = h2


def _mlp_tc(x, parts, Wa, ba, g, be, Wb, bb, relu_out):
    return pl.pallas_call(
        functools.partial(_mlp_body, relu_out=relu_out),
        out_shape=jax.ShapeDtypeStruct((N, C), jnp.float32),
    )(x, parts, Wa.T, ba.reshape(1, C), g.reshape(1, C), be.reshape(1, C),
      Wb.T, bb.reshape(1, C))


def _pool_body(x1_ref, x2_ref, x3_ref, batch_ref, w1_ref, w2_ref, w3_ref,
               blin_ref, out_ref):
    gids = lax.broadcasted_iota(jnp.int32, (N, NG), 1)
    onehot = jnp.where(gids == batch_ref[...], 1.0, 0.0).astype(jnp.float32)
    counts = jnp.sum(onehot, axis=0, keepdims=True)
    inv = 1.0 / jnp.maximum(counts, 1.0)

    dn = (((0,), (0,)), ((), ()))

    def seg(x_ref, w_ref):
        sums = lax.dot_general(onehot, x_ref[...], dn,
                               preferred_element_type=jnp.float32)
        pooled = sums * inv.reshape(NG, 1)
        return jnp.dot(pooled, w_ref[...], preferred_element_type=jnp.float32)

    out = seg(x1_ref, w1_ref) + seg(x2_ref, w2_ref) + seg(x3_ref, w3_ref)
    out_ref[...] = out + blin_ref[...]


def _pool_tc(x1, x2, x3, batch, Wlin, blin):
    w1 = Wlin[:, :C].T
    w2 = Wlin[:, C:2 * C].T
    w3 = Wlin[:, 2 * C:].T
    return pl.pallas_call(
        _pool_body,
        out_shape=jax.ShapeDtypeStruct((NG, C), jnp.float32),
    )(x1, x2, x3, batch.reshape(N, 1), w1, w2, w3, blin.reshape(1, C))


def kernel(x, edge_index, batch, W1a, b1a, g1, be1, W1b, b1b, W2a, b2a, g2,
           be2, W2b, b2b, W3a, b3a, g3, be3, W3b, b3b, Wlin, blin):
    src = edge_index[0]
    dst = edge_index[1]
    zeros = jnp.zeros((N, C), dtype=jnp.float32)

    seg = _segment_sum_sc()
    p1 = seg(x, src, dst, zeros)
    x1 = _mlp_tc(x, p1, W1a, b1a, g1, be1, W1b, b1b, relu_out=True)
    p2 = seg(x1, src, dst, zeros)
    x2 = _mlp_tc(x1, p2, W2a, b2a, g2, be2, W2b, b2b, relu_out=True)
    p3 = seg(x2, src, dst, zeros)
    x3 = _mlp_tc(x2, p3, W3a, b3a, g3, be3, W3b, b3b, relu_out=False)

    return _pool_tc(x1, x2, x3, batch, Wlin, blin)


# speedup vs baseline: 2.4489x; 1.0014x over previous
import functools

import jax
import jax.numpy as jnp
from jax import lax
from jax.experimental import pallas as pl
from jax.experimental.pallas import tpu as pltpu
from jax.experimental.pallas import tpu_sc as plsc

N = 10000
E = 320000
C = 128
NG = 64

NC = 2
NS = 16
NW = NC * NS
K = 80
EPW = E // NW
NCHUNK = EPW // K
NPAIR = (NCHUNK - 1) // 2
RPT = 624
TAIL = N - NS * RPT

def _segment_sum_body(x_hbm, src_hbm, dst_hbm, zeros_hbm, out_hbm,
                      sidx0, sidx1, didx0, didx1, rows0, rows1, acc,
                      isem0, isem1, gsem0, gsem1):
    cid = lax.axis_index("c")
    sid = lax.axis_index("s")
    wid = sid * NC + cid

    pltpu.sync_copy(zeros_hbm.at[pl.ds(sid * RPT, RPT)],
                    acc.at[pl.ds(sid * RPT, RPT)])

    @pl.when(sid == 0)
    def _():
        pltpu.sync_copy(zeros_hbm.at[pl.ds(NS * RPT, TAIL)],
                        acc.at[pl.ds(NS * RPT, TAIL)])

    plsc.subcore_barrier()

    base = wid * EPW

    def iload(j, sbuf, dbuf, sem):
        off = jnp.minimum(base + j * K, E - K)
        pltpu.async_copy(src_hbm.at[pl.ds(off, K)], sbuf, sem)
        pltpu.async_copy(dst_hbm.at[pl.ds(off, K)], dbuf, sem)

    def iwait(sbuf, dbuf, sem):
        pltpu.make_async_copy(src_hbm.at[pl.ds(0, K)], sbuf, sem).wait()
        pltpu.make_async_copy(dst_hbm.at[pl.ds(0, K)], dbuf, sem).wait()

    def gather(sbuf, buf, sem):
        pltpu.async_copy(x_hbm.at[sbuf], buf, sem)

    def gwait(buf, sem):
        pltpu.make_async_copy(x_hbm.at[pl.ds(0, K)], buf, sem).wait()

    def scatter_add(buf, dbuf):
        pltpu.sync_copy(buf, acc.at[dbuf], add=True)

    iload(0, sidx0, didx0, isem0)
    iwait(sidx0, didx0, isem0)
    gather(sidx0, rows0, gsem0)
    iload(1, sidx1, didx1, isem1)

    def halfstep(j, scur, dcur, rcur, isemc, gsemc, soth, doth, roth,
                 isemo, gsemo):
        iwait(soth, doth, isemo)
        gather(soth, roth, gsemo)
        gwait(rcur, gsemc)
        scatter_add(rcur, dcur)
        iload(j + 2, scur, dcur, isemc)

    def outer(i, carry):
        j = 2 * i
        halfstep(j, sidx0, didx0, rows0, isem0, gsem0,
                 sidx1, didx1, rows1, isem1, gsem1)
        halfstep(j + 1, sidx1, didx1, rows1, isem1, gsem1,
                 sidx0, didx0, rows0, isem0, gsem0)
        return carry

    lax.fori_loop(0, NPAIR, outer, 0)
    gwait(rows0, gsem0)
    scatter_add(rows0, didx0)
    iwait(sidx1, didx1, isem1)
    plsc.subcore_barrier()

    pltpu.sync_copy(acc.at[pl.ds(sid * RPT, RPT)],
                    out_hbm.at[cid].at[pl.ds(sid * RPT, RPT)])

    @pl.when(sid == 0)
    def _():
        pltpu.sync_copy(acc.at[pl.ds(NS * RPT, TAIL)],
                        out_hbm.at[cid].at[pl.ds(NS * RPT, TAIL)])


@functools.cache
def _segment_sum_sc():
    mesh = plsc.VectorSubcoreMesh(core_axis_name="c", subcore_axis_name="s",
                                  num_cores=NC, num_subcores=NS)
    return pl.kernel(
        _segment_sum_body,
        out_type=jax.ShapeDtypeStruct((NC, N, C), jnp.float32),
        mesh=mesh,
        scratch_types=[
            pltpu.VMEM((K,), jnp.int32),
            pltpu.VMEM((K,), jnp.int32),
            pltpu.VMEM((K,), jnp.int32),
            pltpu.VMEM((K,), jnp.int32),
            pltpu.VMEM((K, C), jnp.float32),
            pltpu.VMEM((K, C), jnp.float32),
            pltpu.VMEM_SHARED((N, C), jnp.float32),
            pltpu.SemaphoreType.DMA,
            pltpu.SemaphoreType.DMA,
            pltpu.SemaphoreType.DMA,
            pltpu.SemaphoreType.DMA,
        ],
    )


def _mlp_body(x_ref, parts_ref, wat_ref, ba_ref, g_ref, be_ref, wbt_ref,
              bb_ref, out_ref, *, relu_out):
    h = x_ref[...] + parts_ref[0] + parts_ref[1]
    h1 = jnp.dot(h, wat_ref[...], preferred_element_type=jnp.float32)
    h1 = h1 + ba_ref[...]
    m = jnp.mean(h1, axis=0, keepdims=True)
    d = h1 - m
    v = jnp.mean(d * d, axis=0, keepdims=True)
    hb = g_ref[...] * d * lax.rsqrt(v + 1e-5) + be_ref[...]
    hb = jnp.maximum(hb, 0.0)
    h2 = jnp.dot(hb, wbt_ref[...], preferred_element_type=jnp.float32)
    h2 = h2 + bb_ref[...]
    if relu_out:
        h2 = jnp.maximum(h2, 0.0)
    out_ref[...] = h2


def _mlp_tc(x, parts, Wa, ba, g, be, Wb, bb, relu_out):
    return pl.pallas_call(
        functools.partial(_mlp_body, relu_out=relu_out),
        out_shape=jax.ShapeDtypeStruct((N, C), jnp.float32),
    )(x, parts, Wa.T, ba.reshape(1, C), g.reshape(1, C), be.reshape(1, C),
      Wb.T, bb.reshape(1, C))


def _pool_body(x1_ref, x2_ref, x3_ref, batch_ref, w1_ref, w2_ref, w3_ref,
               blin_ref, out_ref):
    gids = lax.broadcasted_iota(jnp.int32, (N, NG), 1)
    onehot = jnp.where(gids == batch_ref[...], 1.0, 0.0).astype(jnp.float32)
    counts = jnp.sum(onehot, axis=0, keepdims=True)
    inv = 1.0 / jnp.maximum(counts, 1.0)

    dn = (((0,), (0,)), ((), ()))

    def seg(x_ref, w_ref):
        sums = lax.dot_general(onehot, x_ref[...], dn,
                               preferred_element_type=jnp.float32)
        pooled = sums * inv.reshape(NG, 1)
        return jnp.dot(pooled, w_ref[...], preferred_element_type=jnp.float32)

    out = seg(x1_ref, w1_ref) + seg(x2_ref, w2_ref) + seg(x3_ref, w3_ref)
    out_ref[...] = out + blin_ref[...]


def _pool_tc(x1, x2, x3, batch, Wlin, blin):
    w1 = Wlin[:, :C].T
    w2 = Wlin[:, C:2 * C].T
    w3 = Wlin[:, 2 * C:].T
    return pl.pallas_call(
        _pool_body,
        out_shape=jax.ShapeDtypeStruct((NG, C), jnp.float32),
    )(x1, x2, x3, batch.reshape(N, 1), w1, w2, w3, blin.reshape(1, C))


def kernel(x, edge_index, batch, W1a, b1a, g1, be1, W1b, b1b, W2a, b2a, g2,
           be2, W2b, b2b, W3a, b3a, g3, be3, W3b, b3b, Wlin, blin):
    src = edge_index[0]
    dst = edge_index[1]
    zeros = jnp.zeros((N, C), dtype=jnp.float32)

    seg = _segment_sum_sc()
    p1 = seg(x, src, dst, zeros)
    x1 = _mlp_tc(x, p1, W1a, b1a, g1, be1, W1b, b1b, relu_out=True)
    p2 = seg(x1, src, dst, zeros)
    x2 = _mlp_tc(x1, p2, W2a, b2a, g2, be2, W2b, b2b, relu_out=True)
    p3 = seg(x2, src, dst, zeros)
    x3 = _mlp_tc(x2, p3, W3a, b3a, g3, be3, W3b, b3b, relu_out=False)

    return _pool_tc(x1, x2, x3, batch, Wlin, blin)
